# baseline (speedup 1.0000x reference)
import functools

import jax
import jax.numpy as jnp
from jax import lax
from jax.experimental import pallas as pl
from jax.experimental.pallas import tpu as pltpu
from jax.experimental.pallas import tpu_sc as plsc

B = 16384
D = 64
NC = 2
NS = 16
NW = NC * NS
BPW = B // NW
L = 16
NGRP = BPW // L
V = 1000000
BIAS_ROWS = 7813
TBLK = 512


def _relayout_body(xt_ref, out_ref):
  y = xt_ref[...].T.reshape(TBLK // 2, 2, D)
  out_ref[:, 0:64] = y[:, 0, :]
  out_ref[:, 64:128] = y[:, 1, :]


def _to_linear(xt):
  return pl.pallas_call(
      _relayout_body,
      grid=((V + TBLK - 1) // TBLK,),
      in_specs=[pl.BlockSpec((D, TBLK), lambda j: (0, j))],
      out_specs=pl.BlockSpec((TBLK // 2, 128), lambda j: (j, 0)),
      out_shape=jax.ShapeDtypeStruct((V // 2, 128), jnp.float32),
  )(xt)


def _sc_body(src_hbm, dst_hbm, uemb, ubias_p, iemb, ibias_p,
             mean_hbm, out_hbm,
             src_v, dst_v, u16, v16, ub16, vb16, out_v, mean_v, sem):
  wid = lax.axis_index("s") * NC + lax.axis_index("c")
  base = wid * BPW

  pltpu.sync_copy(src_hbm.at[pl.ds(base, BPW)], src_v)
  pltpu.sync_copy(dst_hbm.at[pl.ds(base, BPW)], dst_v)
  pltpu.sync_copy(mean_hbm, mean_v)

  lanes = lax.iota(jnp.int32, L)
  m1 = jnp.full((L,), 1, jnp.int32)
  m127 = jnp.full((L,), 127, jnp.int32)
  mean_vec = mean_v[...]

  def group(g, carry):
    s_idx = src_v[pl.ds(g * L, L)]
    d_idx = dst_v[pl.ds(g * L, L)]
    cps = [
        pltpu.async_copy(uemb.at[lax.shift_right_logical(s_idx, 1)], u16, sem),
        pltpu.async_copy(iemb.at[lax.shift_right_logical(d_idx, 1)], v16, sem),
        pltpu.async_copy(ubias_p.at[lax.shift_right_logical(s_idx, 7)], ub16, sem),
        pltpu.async_copy(ibias_p.at[lax.shift_right_logical(d_idx, 7)], vb16, sem),
    ]
    for cp in cps:
      cp.wait()
    s_half = lax.bitwise_and(s_idx, m1) * 64
    d_half = lax.bitwise_and(d_idx, m1) * 64
    acc = jnp.zeros((L,), jnp.float32)
    for d in range(D):
      dvec = jnp.full((L,), d, jnp.int32)
      u = plsc.load_gather(u16, [lanes, s_half + dvec])
      v = plsc.load_gather(v16, [lanes, d_half + dvec])
      acc = acc + u * v
    ub = plsc.load_gather(ub16, [lanes, lax.bitwise_and(s_idx, m127)])
    vb = plsc.load_gather(vb16, [lanes, lax.bitwise_and(d_idx, m127)])
    out_v[pl.ds(g * L, L)] = acc + ub + vb + mean_vec
    return carry

  lax.fori_loop(0, NGRP, group, 0)

  pltpu.sync_copy(out_v, out_hbm.at[pl.ds(base, BPW)])


@jax.jit
def kernel(src, dst, user_emb, user_bias, item_emb, item_bias, mean):
  src1 = src.astype(jnp.int32)
  dst1 = dst.astype(jnp.int32)
  u_lin = _to_linear(user_emb.T)
  i_lin = _to_linear(item_emb.T)
  ubias_p = jnp.pad(user_bias.reshape(-1), (0, BIAS_ROWS * 128 - V)).reshape(BIAS_ROWS, 128)
  ibias_p = jnp.pad(item_bias.reshape(-1), (0, BIAS_ROWS * 128 - V)).reshape(BIAS_ROWS, 128)
  mean16 = jnp.broadcast_to(mean.astype(jnp.float32), (L,))
  mesh = plsc.VectorSubcoreMesh(core_axis_name="c", subcore_axis_name="s")
  run = functools.partial(
      pl.kernel,
      out_type=jax.ShapeDtypeStruct((B,), jnp.float32),
      mesh=mesh,
      compiler_params=pltpu.CompilerParams(
          needs_layout_passes=False, use_tc_tiling_on_sc=True),
      scratch_types=[
          pltpu.VMEM((BPW,), jnp.int32),
          pltpu.VMEM((BPW,), jnp.int32),
          pltpu.VMEM((L, 128), jnp.float32),
          pltpu.VMEM((L, 128), jnp.float32),
          pltpu.VMEM((L, 128), jnp.float32),
          pltpu.VMEM((L, 128), jnp.float32),
          pltpu.VMEM((BPW,), jnp.float32),
          pltpu.VMEM((L,), jnp.float32),
          pltpu.SemaphoreType.DMA,
      ],
  )(_sc_body)
  return run(src1, dst1, u_lin, ubias_p, i_lin, ibias_p, mean16)


# speedup vs baseline: 2.3221x; 2.3221x over previous
import functools

import jax
import jax.numpy as jnp
from jax import lax
from jax.experimental import pallas as pl
from jax.experimental.pallas import tpu as pltpu
from jax.experimental.pallas import tpu_sc as plsc

B = 16384
D = 64
NC = 2
NS = 16
NW = NC * NS
BPW = B // NW
L = 16
NGRP = BPW // L
V = 1000000
BIAS_ROWS = 7813
TBLK = 512


def _relayout_body(xt_ref, out_ref):
  y = xt_ref[...].T.reshape(TBLK // 2, 2, D)
  out_ref[:, 0:64] = y[:, 0, :]
  out_ref[:, 64:128] = y[:, 1, :]


def _to_linear(xt):
  return pl.pallas_call(
      _relayout_body,
      grid=((V + TBLK - 1) // TBLK,),
      in_specs=[pl.BlockSpec((D, TBLK), lambda j: (0, j))],
      out_specs=pl.BlockSpec((TBLK // 2, 128), lambda j: (j, 0)),
      out_shape=jax.ShapeDtypeStruct((V // 2, 128), jnp.float32),
  )(xt)


def _sc_body(src_hbm, dst_hbm, uemb, ubias_p, iemb, ibias_p,
             mean_hbm, out_hbm,
             src_v, dst_v, u16, v16, ub16, vb16, out_v, mean_v, sem):
  wid = lax.axis_index("s") * NC + lax.axis_index("c")
  base = wid * BPW

  pltpu.sync_copy(src_hbm.at[pl.ds(base, BPW)], src_v)
  pltpu.sync_copy(dst_hbm.at[pl.ds(base, BPW)], dst_v)
  pltpu.sync_copy(mean_hbm, mean_v)

  lanes = lax.iota(jnp.int32, L)
  m1 = jnp.full((L,), 1, jnp.int32)
  m127 = jnp.full((L,), 127, jnp.int32)
  mean_vec = mean_v[...]

  def group(g, carry):
    s_idx = src_v[pl.ds(g * L, L)]
    d_idx = dst_v[pl.ds(g * L, L)]
    cps = [
        pltpu.async_copy(uemb.at[lax.shift_right_logical(s_idx, 1)], u16, sem),
        pltpu.async_copy(iemb.at[lax.shift_right_logical(d_idx, 1)], v16, sem),
        pltpu.async_copy(ubias_p.at[lax.shift_right_logical(s_idx, 7)], ub16, sem),
        pltpu.async_copy(ibias_p.at[lax.shift_right_logical(d_idx, 7)], vb16, sem),
    ]
    for cp in cps:
      cp.wait()
    s_half = lax.bitwise_and(s_idx, m1) * 64
    d_half = lax.bitwise_and(d_idx, m1) * 64
    acc = jnp.zeros((L,), jnp.float32)
    for d in range(D):
      dvec = jnp.full((L,), d, jnp.int32)
      u = plsc.load_gather(u16, [lanes, s_half + dvec])
      v = plsc.load_gather(v16, [lanes, d_half + dvec])
      acc = acc + u * v
    ub = plsc.load_gather(ub16, [lanes, lax.bitwise_and(s_idx, m127)])
    vb = plsc.load_gather(vb16, [lanes, lax.bitwise_and(d_idx, m127)])
    out_v[pl.ds(g * L, L)] = acc + ub + vb + mean_vec
    return carry

  lax.fori_loop(0, NGRP, group, 0)

  pltpu.sync_copy(out_v, out_hbm.at[pl.ds(base, BPW)])


@jax.jit
def kernel(src, dst, user_emb, user_bias, item_emb, item_bias, mean):
  src1 = src.astype(jnp.int32)
  dst1 = dst.astype(jnp.int32)
  u_lin = user_emb.reshape(V // 2, 128)
  i_lin = item_emb.reshape(V // 2, 128)
  ubias_p = jnp.pad(user_bias.reshape(-1), (0, BIAS_ROWS * 128 - V)).reshape(BIAS_ROWS, 128)
  ibias_p = jnp.pad(item_bias.reshape(-1), (0, BIAS_ROWS * 128 - V)).reshape(BIAS_ROWS, 128)
  mean16 = jnp.broadcast_to(mean.astype(jnp.float32), (L,))
  mesh = plsc.VectorSubcoreMesh(core_axis_name="c", subcore_axis_name="s")
  run = functools.partial(
      pl.kernel,
      out_type=jax.ShapeDtypeStruct((B,), jnp.float32),
      mesh=mesh,
      compiler_params=pltpu.CompilerParams(
          needs_layout_passes=False, use_tc_tiling_on_sc=True),
      scratch_types=[
          pltpu.VMEM((BPW,), jnp.int32),
          pltpu.VMEM((BPW,), jnp.int32),
          pltpu.VMEM((L, 128), jnp.float32),
          pltpu.VMEM((L, 128), jnp.float32),
          pltpu.VMEM((L, 128), jnp.float32),
          pltpu.VMEM((L, 128), jnp.float32),
          pltpu.VMEM((BPW,), jnp.float32),
          pltpu.VMEM((L,), jnp.float32),
          pltpu.SemaphoreType.DMA,
      ],
  )(_sc_body)
  return run(src1, dst1, u_lin, ubias_p, i_lin, ibias_p, mean16)


# speedup vs baseline: 2.4663x; 1.0621x over previous
import functools

import jax
import jax.numpy as jnp
from jax import lax
from jax.experimental import pallas as pl
from jax.experimental.pallas import tpu as pltpu
from jax.experimental.pallas import tpu_sc as plsc

B = 16384
D = 64
NC = 2
NS = 16
NW = NC * NS
BPW = B // NW
L = 16
NGRP = BPW // L
V = 1000000
BIAS_ROWS = 7813
TBLK = 512


def _relayout_body(xt_ref, out_ref):
  y = xt_ref[...].T.reshape(TBLK // 2, 2, D)
  out_ref[:, 0:64] = y[:, 0, :]
  out_ref[:, 64:128] = y[:, 1, :]


def _to_linear(xt):
  return pl.pallas_call(
      _relayout_body,
      grid=((V + TBLK - 1) // TBLK,),
      in_specs=[pl.BlockSpec((D, TBLK), lambda j: (0, j))],
      out_specs=pl.BlockSpec((TBLK // 2, 128), lambda j: (j, 0)),
      out_shape=jax.ShapeDtypeStruct((V // 2, 128), jnp.float32),
  )(xt)


def _sc_body(src_hbm, dst_hbm, uemb, ubias_p, iemb, ibias_p,
             mean_hbm, out_hbm,
             src_v, dst_v, u16, v16, ub16, vb16, out_v, mean_v, sem):
  wid = lax.axis_index("s") * NC + lax.axis_index("c")
  base = wid * BPW

  pltpu.sync_copy(src_hbm.at[pl.ds(base, BPW)], src_v)
  pltpu.sync_copy(dst_hbm.at[pl.ds(base, BPW)], dst_v)
  pltpu.sync_copy(mean_hbm, mean_v)

  lanes = lax.iota(jnp.int32, L)
  m127 = jnp.full((L,), 127, jnp.int32)
  mean_vec = mean_v[...]

  def group(g, carry):
    s_idx = src_v[pl.ds(g * L, L)]
    d_idx = dst_v[pl.ds(g * L, L)]
    cps = [
        pltpu.async_copy(uemb.at[s_idx], u16, sem),
        pltpu.async_copy(iemb.at[d_idx], v16, sem),
        pltpu.async_copy(ubias_p.at[lax.shift_right_logical(s_idx, 7)], ub16, sem),
        pltpu.async_copy(ibias_p.at[lax.shift_right_logical(d_idx, 7)], vb16, sem),
    ]
    for cp in cps:
      cp.wait()
    acc = jnp.zeros((L,), jnp.float32)
    for d in range(D):
      dvec = jnp.full((L,), d, jnp.int32)
      u = plsc.load_gather(u16, [lanes, dvec])
      v = plsc.load_gather(v16, [lanes, dvec])
      acc = acc + u * v
    ub = plsc.load_gather(ub16, [lanes, lax.bitwise_and(s_idx, m127)])
    vb = plsc.load_gather(vb16, [lanes, lax.bitwise_and(d_idx, m127)])
    out_v[pl.ds(g * L, L)] = acc + ub + vb + mean_vec
    return carry

  lax.fori_loop(0, NGRP, group, 0)

  pltpu.sync_copy(out_v, out_hbm.at[pl.ds(base, BPW)])


@jax.jit
def kernel(src, dst, user_emb, user_bias, item_emb, item_bias, mean):
  src1 = src.astype(jnp.int32)
  dst1 = dst.astype(jnp.int32)
  u_lin = jnp.pad(user_emb, ((0, 0), (0, 128 - D)))
  i_lin = jnp.pad(item_emb, ((0, 0), (0, 128 - D)))
  ubias_p = jnp.pad(user_bias.reshape(-1), (0, BIAS_ROWS * 128 - V)).reshape(BIAS_ROWS, 128)
  ibias_p = jnp.pad(item_bias.reshape(-1), (0, BIAS_ROWS * 128 - V)).reshape(BIAS_ROWS, 128)
  mean16 = jnp.broadcast_to(mean.astype(jnp.float32), (L,))
  mesh = plsc.VectorSubcoreMesh(core_axis_name="c", subcore_axis_name="s")
  run = functools.partial(
      pl.kernel,
      out_type=jax.ShapeDtypeStruct((B,), jnp.float32),
      mesh=mesh,
      compiler_params=pltpu.CompilerParams(
          needs_layout_passes=False, use_tc_tiling_on_sc=True),
      scratch_types=[
          pltpu.VMEM((BPW,), jnp.int32),
          pltpu.VMEM((BPW,), jnp.int32),
          pltpu.VMEM((L, 128), jnp.float32),
          pltpu.VMEM((L, 128), jnp.float32),
          pltpu.VMEM((L, 128), jnp.float32),
          pltpu.VMEM((L, 128), jnp.float32),
          pltpu.VMEM((BPW,), jnp.float32),
          pltpu.VMEM((L,), jnp.float32),
          pltpu.SemaphoreType.DMA,
      ],
  )(_sc_body)
  return run(src1, dst1, u_lin, ubias_p, i_lin, ibias_p, mean16)


# speedup vs baseline: 3.0630x; 1.2419x over previous
import functools

import jax
import jax.numpy as jnp
from jax import lax
from jax.experimental import pallas as pl
from jax.experimental.pallas import tpu as pltpu
from jax.experimental.pallas import tpu_sc as plsc

B = 16384
D = 64
NC = 2
NS = 16
NW = NC * NS
BPW = B // NW
L = 16
NGRP = BPW // L
V = 1000000
BIAS_ROWS = 7813


def _sc_body(src_hbm, dst_hbm, uemb, ubias_p, iemb, ibias_p,
             mean_hbm, out_hbm,
             src_v, dst_v, u8, v8, ub16, vb16, out_v, mean_v, sem):
  wid = lax.axis_index("s") * NC + lax.axis_index("c")
  base = wid * BPW

  pltpu.sync_copy(src_hbm.at[pl.ds(base, BPW)], src_v)
  pltpu.sync_copy(dst_hbm.at[pl.ds(base, BPW)], dst_v)
  pltpu.sync_copy(mean_hbm, mean_v)

  lanes = lax.iota(jnp.int32, L)
  m7 = jnp.full((L,), 7, jnp.int32)
  m127 = jnp.full((L,), 127, jnp.int32)
  mean_vec = mean_v[...]

  def group(g, carry):
    s_idx = src_v[pl.ds(g * L, L)]
    d_idx = dst_v[pl.ds(g * L, L)]
    cps = []
    s_base = s_idx - lax.bitwise_and(s_idx, m7)
    d_base = d_idx - lax.bitwise_and(d_idx, m7)
    for e in range(L):
      rb = pl.multiple_of(s_base[e], 8)
      cps.append(pltpu.async_copy(uemb.at[pl.ds(rb, 8)], u8.at[e], sem))
      qb = pl.multiple_of(d_base[e], 8)
      cps.append(pltpu.async_copy(iemb.at[pl.ds(qb, 8)], v8.at[e], sem))
    cps.append(
        pltpu.async_copy(ubias_p.at[lax.shift_right_logical(s_idx, 7)], ub16, sem))
    cps.append(
        pltpu.async_copy(ibias_p.at[lax.shift_right_logical(d_idx, 7)], vb16, sem))
    for cp in cps:
      cp.wait()
    s_sub = lax.bitwise_and(s_idx, m7)
    d_sub = lax.bitwise_and(d_idx, m7)
    acc = jnp.zeros((L,), jnp.float32)
    for d in range(D):
      dvec = jnp.full((L,), d, jnp.int32)
      u = plsc.load_gather(u8, [lanes, s_sub, dvec])
      v = plsc.load_gather(v8, [lanes, d_sub, dvec])
      acc = acc + u * v
    ub = plsc.load_gather(ub16, [lanes, lax.bitwise_and(s_idx, m127)])
    vb = plsc.load_gather(vb16, [lanes, lax.bitwise_and(d_idx, m127)])
    out_v[pl.ds(g * L, L)] = acc + ub + vb + mean_vec
    return carry

  lax.fori_loop(0, NGRP, group, 0)

  pltpu.sync_copy(out_v, out_hbm.at[pl.ds(base, BPW)])


@jax.jit
def kernel(src, dst, user_emb, user_bias, item_emb, item_bias, mean):
  src1 = src.astype(jnp.int32)
  dst1 = dst.astype(jnp.int32)
  ubias_p = jnp.pad(user_bias.reshape(-1), (0, BIAS_ROWS * 128 - V)).reshape(BIAS_ROWS, 128)
  ibias_p = jnp.pad(item_bias.reshape(-1), (0, BIAS_ROWS * 128 - V)).reshape(BIAS_ROWS, 128)
  mean16 = jnp.broadcast_to(mean.astype(jnp.float32), (L,))
  mesh = plsc.VectorSubcoreMesh(core_axis_name="c", subcore_axis_name="s")
  run = functools.partial(
      pl.kernel,
      out_type=jax.ShapeDtypeStruct((B,), jnp.float32),
      mesh=mesh,
      compiler_params=pltpu.CompilerParams(
          needs_layout_passes=False, use_tc_tiling_on_sc=True),
      scratch_types=[
          pltpu.VMEM((BPW,), jnp.int32),
          pltpu.VMEM((BPW,), jnp.int32),
          pltpu.VMEM((L, 8, D), jnp.float32),
          pltpu.VMEM((L, 8, D), jnp.float32),
          pltpu.VMEM((L, 128), jnp.float32),
          pltpu.VMEM((L, 128), jnp.float32),
          pltpu.VMEM((BPW,), jnp.float32),
          pltpu.VMEM((L,), jnp.float32),
          pltpu.SemaphoreType.DMA,
      ],
  )(_sc_body)
  return run(src1, dst1, user_emb, ubias_p, item_emb, ibias_p, mean16)
